# baseline (speedup 1.0000x reference)
import functools

import jax
import jax.numpy as jnp
from jax import lax
from jax.experimental import pallas as pl
from jax.experimental.pallas import tpu as pltpu, tpu_sc as plsc

N_NODES = 10000
N_EDGES = 320000
D_FEAT = 128
HIDDEN = 128
N_CLASSES = 40

NC = 2
NS = 16
LANES = 16

EDGES_PER_TILE = N_EDGES // (NC * NS)
KBLK = 32
NDEPTH = 4
NB_CH = 32
NCHUNK = 10
EPT_PAD = NCHUNK * NB_CH * KBLK
N_PAD = N_NODES + 8


@functools.cache
def _mesh():
    return plsc.VectorSubcoreMesh(
        core_axis_name="c", subcore_axis_name="s", num_cores=NC, num_subcores=NS
    )


def _deg_body(dst_hbm, out_hbm, dst_v, hist_v):
    c = lax.axis_index("c")
    s = lax.axis_index("s")
    w = c * NS + s
    pltpu.sync_copy(dst_hbm.at[w], dst_v)
    zeros = jnp.zeros((LANES,), jnp.float32)

    def zbody(i, _):
        hist_v[pl.ds(i * LANES, LANES)] = zeros
        return 0

    lax.fori_loop(0, N_NODES // LANES, zbody, 0)
    ones = jnp.ones((LANES,), jnp.float32)

    def body(i, _):
        idx = dst_v[pl.ds(i * LANES, LANES)]
        plsc.addupdate_scatter(hist_v, [idx], ones)
        return 0

    lax.fori_loop(0, EDGES_PER_TILE // LANES, body, 0)
    pltpu.sync_copy(hist_v, out_hbm.at[w])


@functools.cache
def _deg_call():
    return pl.kernel(
        _deg_body,
        out_type=jax.ShapeDtypeStruct((NC * NS, N_NODES), jnp.float32),
        mesh=_mesh(),
        scratch_types=[
            pltpu.VMEM((EDGES_PER_TILE,), jnp.int32),
            pltpu.VMEM((N_NODES,), jnp.float32),
        ],
        compiler_params=pltpu.CompilerParams(needs_layout_passes=False),
    )


RCHUNK = 624
RLAST = N_NODES - (NS - 1) * RCHUNK


def _stage(s, src_view, dst_view):
    r0 = pl.multiple_of(s * RCHUNK, 8)

    @pl.when(s < NS - 1)
    def _():
        pltpu.sync_copy(src_view.at[pl.ds(r0, RCHUNK)],
                        dst_view.at[pl.ds(r0, RCHUNK)])

    @pl.when(s == NS - 1)
    def _():
        pltpu.sync_copy(src_view.at[pl.ds((NS - 1) * RCHUNK, RLAST)],
                        dst_view.at[pl.ds((NS - 1) * RCHUNK, RLAST)])


def _agg_body(src_hbm, dst_hbm, y_hbm, out_hbm, src_v, dst_v,
              gbuf0, gbuf1, gbuf2, gbuf3, acc_sh,
              gsem0, gsem1, gsem2, gsem3, ssem0, ssem1, ssem2, ssem3):
    c = lax.axis_index("c")
    s = lax.axis_index("s")
    gbufs = (gbuf0, gbuf1, gbuf2, gbuf3)
    gsems = (gsem0, gsem1, gsem2, gsem3)
    ssems = (ssem0, ssem1, ssem2, ssem3)
    _stage(s, y_hbm, acc_sh)
    plsc.subcore_barrier()

    def g_start(jv, b):
        pltpu.async_copy(y_hbm.at[src_v.at[jv]], gbufs[b], gsems[b])

    def g_wait(b):
        pltpu.make_async_copy(y_hbm.at[src_v.at[0]], gbufs[b], gsems[b]).wait()

    def s_start(jv, b):
        pltpu.async_copy(gbufs[b], acc_sh.at[dst_v.at[jv]], ssems[b], add=True)

    def s_wait(b):
        pltpu.make_async_copy(gbufs[b], acc_sh.at[dst_v.at[0]], ssems[b]).wait()

    def chunk(ch, _):
        pltpu.sync_copy(src_hbm.at[c, s, ch], src_v)
        pltpu.sync_copy(dst_hbm.at[c, s, ch], dst_v)
        for b in range(NDEPTH):
            g_start(b, b)

        def group(g, _):
            j = NDEPTH * g
            for b in range(NDEPTH):
                g_wait(b)
                s_start(j + b, b)

                @pl.when(g < NB_CH // NDEPTH - 1)
                def _():
                    s_wait(b)
                    g_start(j + b + NDEPTH, b)

            return 0

        lax.fori_loop(0, NB_CH // NDEPTH, group, 0)
        for b in range(NDEPTH):
            s_wait(b)
        return 0

    lax.fori_loop(0, NCHUNK, chunk, 0)
    plsc.subcore_barrier()
    _stage(s, acc_sh, out_hbm.at[c])


@functools.cache
def _agg_call():
    return pl.kernel(
        _agg_body,
        out_type=jax.ShapeDtypeStruct((NC, N_NODES, D_FEAT), jnp.float32),
        mesh=_mesh(),
        scratch_types=(
            [pltpu.VMEM((NB_CH, KBLK), jnp.int32)] * 2
            + [pltpu.VMEM((KBLK, D_FEAT), jnp.float32)] * NDEPTH
            + [pltpu.MemorySpace.VMEM_SHARED((N_PAD, D_FEAT), jnp.float32)]
            + [pltpu.SemaphoreType.DMA] * (2 * NDEPTH)
        ),
    )


MBLK = 1000
GRID = N_NODES // MBLK


def _mm1_body(deg_ref, x_ref, w_ref, y_ref, dinv_ref):
    deg = jnp.sum(deg_ref[...], axis=0) + 1.0
    dinv = lax.rsqrt(deg)
    xw = jnp.dot(x_ref[...], w_ref[...], preferred_element_type=jnp.float32)
    y_ref[...] = xw * dinv
    dinv_ref[...] = dinv


def _mid_body(agg_ref, y_ref, dinv_ref, b1_ref, w_ref, out_ref):
    a = agg_ref[0] + agg_ref[1] - y_ref[...]
    dinv = dinv_ref[...]
    h = jnp.maximum(a * dinv + b1_ref[...], 0.0)
    out_ref[...] = jnp.dot(h, w_ref[...], preferred_element_type=jnp.float32) * dinv


def _head_body(agg_ref, y_ref, dinv_ref, b2_ref, w3_ref, b3_ref, out_ref):
    a = agg_ref[0] + agg_ref[1] - y_ref[...]
    h = jnp.maximum(a * dinv_ref[...] + b2_ref[...], 0.0)
    out_ref[...] = (
        jnp.dot(h, w3_ref[...], preferred_element_type=jnp.float32) + b3_ref[...]
    )


def _mm1(degs, x, W1):
    return pl.pallas_call(
        _mm1_body,
        grid=(GRID,),
        in_specs=[
            pl.BlockSpec((NC * NS, MBLK, 1), lambda i: (0, i, 0)),
            pl.BlockSpec((MBLK, D_FEAT), lambda i: (i, 0)),
            pl.BlockSpec((D_FEAT, HIDDEN), lambda i: (0, 0)),
        ],
        out_specs=[
            pl.BlockSpec((MBLK, HIDDEN), lambda i: (i, 0)),
            pl.BlockSpec((MBLK, 1), lambda i: (i, 0)),
        ],
        out_shape=[
            jax.ShapeDtypeStruct((N_NODES, HIDDEN), jnp.float32),
            jax.ShapeDtypeStruct((N_NODES, 1), jnp.float32),
        ],
    )(degs, x, W1)


def _mid(agg1, y1, dinv, b1, W2):
    return pl.pallas_call(
        _mid_body,
        grid=(GRID,),
        in_specs=[
            pl.BlockSpec((NC, MBLK, HIDDEN), lambda i: (0, i, 0)),
            pl.BlockSpec((MBLK, HIDDEN), lambda i: (i, 0)),
            pl.BlockSpec((MBLK, 1), lambda i: (i, 0)),
            pl.BlockSpec((1, HIDDEN), lambda i: (0, 0)),
            pl.BlockSpec((HIDDEN, HIDDEN), lambda i: (0, 0)),
        ],
        out_specs=pl.BlockSpec((MBLK, HIDDEN), lambda i: (i, 0)),
        out_shape=jax.ShapeDtypeStruct((N_NODES, HIDDEN), jnp.float32),
    )(agg1, y1, dinv, b1, W2)


def _head(agg2, y2, dinv, b2, W3, b3):
    return pl.pallas_call(
        _head_body,
        grid=(GRID,),
        in_specs=[
            pl.BlockSpec((NC, MBLK, HIDDEN), lambda i: (0, i, 0)),
            pl.BlockSpec((MBLK, HIDDEN), lambda i: (i, 0)),
            pl.BlockSpec((MBLK, 1), lambda i: (i, 0)),
            pl.BlockSpec((1, HIDDEN), lambda i: (0, 0)),
            pl.BlockSpec((HIDDEN, N_CLASSES), lambda i: (0, 0)),
            pl.BlockSpec((1, N_CLASSES), lambda i: (0, 0)),
        ],
        out_specs=pl.BlockSpec((MBLK, N_CLASSES), lambda i: (i, 0)),
        out_shape=jax.ShapeDtypeStruct((N_NODES, N_CLASSES), jnp.float32),
    )(agg2, y2, dinv, b2, W3, b3)


def kernel(x, edge_index, W1, b1, W2, b2, W3, b3):
    ei = edge_index.astype(jnp.int32)
    pad = ((0, 0), (0, EPT_PAD - EDGES_PER_TILE))
    src = jnp.pad(ei[0].reshape(NC * NS, EDGES_PER_TILE), pad,
                  constant_values=N_NODES).reshape(NC, NS, NCHUNK, NB_CH, KBLK)
    dst = jnp.pad(ei[1].reshape(NC * NS, EDGES_PER_TILE), pad,
                  constant_values=N_NODES).reshape(NC, NS, NCHUNK, NB_CH, KBLK)
    dst_deg = ei[1].reshape(NC * NS, EDGES_PER_TILE)

    degs = _deg_call()(dst_deg)
    y1, dinv = _mm1(degs.reshape(NC * NS, N_NODES, 1), x, W1)
    y1p = jnp.pad(y1, ((0, N_PAD - N_NODES), (0, 0)))
    agg1 = _agg_call()(src, dst, y1p)
    y2 = _mid(agg1, y1, dinv, b1.reshape(1, HIDDEN), W2)
    y2p = jnp.pad(y2, ((0, N_PAD - N_NODES), (0, 0)))
    agg2 = _agg_call()(src, dst, y2p)
    logits = _head(agg2, y2, dinv, b2.reshape(1, HIDDEN), W3,
                   b3.reshape(1, N_CLASSES))
    return logits


# speedup vs baseline: 1.7552x; 1.7552x over previous
import functools

import jax
import jax.numpy as jnp
from jax import lax
from jax.experimental import pallas as pl
from jax.experimental.pallas import tpu as pltpu, tpu_sc as plsc

N_NODES = 10000
N_EDGES = 320000
D_FEAT = 128
HIDDEN = 128
N_CLASSES = 40

NC = 2
NS = 16
LANES = 16

EDGES_PER_TILE = N_EDGES // (NC * NS)
KBLK = 48
NB_CH = 42
NCHUNK = 5
EPT_PAD = NCHUNK * NB_CH * KBLK
N_PAD = N_NODES + NS


@functools.cache
def _mesh():
    return plsc.VectorSubcoreMesh(
        core_axis_name="c", subcore_axis_name="s", num_cores=NC, num_subcores=NS
    )


def _deg_body(dst_hbm, out_hbm, dst_v, hist_v):
    c = lax.axis_index("c")
    s = lax.axis_index("s")
    w = c * NS + s
    pltpu.sync_copy(dst_hbm.at[w], dst_v)
    zeros = jnp.zeros((LANES,), jnp.float32)

    def zbody(i, _):
        hist_v[pl.ds(i * LANES, LANES)] = zeros
        return 0

    lax.fori_loop(0, N_NODES // LANES, zbody, 0)
    ones = jnp.ones((LANES,), jnp.float32)

    def body(i, _):
        idx = dst_v[pl.ds(i * LANES, LANES)]
        plsc.addupdate_scatter(hist_v, [idx], ones)
        return 0

    lax.fori_loop(0, EDGES_PER_TILE // LANES, body, 0)
    pltpu.sync_copy(hist_v, out_hbm.at[w])


@functools.cache
def _deg_call():
    return pl.kernel(
        _deg_body,
        out_type=jax.ShapeDtypeStruct((NC * NS, N_NODES), jnp.float32),
        mesh=_mesh(),
        scratch_types=[
            pltpu.VMEM((EDGES_PER_TILE,), jnp.int32),
            pltpu.VMEM((N_NODES,), jnp.float32),
        ],
        compiler_params=pltpu.CompilerParams(needs_layout_passes=False),
    )


RCHUNK = 624
RLAST = N_NODES - (NS - 1) * RCHUNK


def _stage(s, src_view, dst_view):
    r0 = pl.multiple_of(s * RCHUNK, 8)

    @pl.when(s < NS - 1)
    def _():
        pltpu.sync_copy(src_view.at[pl.ds(r0, RCHUNK)],
                        dst_view.at[pl.ds(r0, RCHUNK)])

    @pl.when(s == NS - 1)
    def _():
        pltpu.sync_copy(src_view.at[pl.ds((NS - 1) * RCHUNK, RLAST)],
                        dst_view.at[pl.ds((NS - 1) * RCHUNK, RLAST)])


def _agg_body(src_hbm, dst_hbm, y_hbm, out_hbm, src_v, dst_v, gbuf0, gbuf1,
              acc_sh, gsem0, gsem1, ssem0, ssem1):
    c = lax.axis_index("c")
    s = lax.axis_index("s")
    _stage(s, y_hbm, acc_sh)
    plsc.subcore_barrier()

    def g_start(jv, buf, sem):
        pltpu.async_copy(y_hbm.at[src_v.at[jv]], buf, sem)

    def g_wait(buf, sem):
        pltpu.make_async_copy(y_hbm.at[src_v.at[0]], buf, sem).wait()

    def s_start(jv, buf, sem):
        pltpu.async_copy(buf, acc_sh.at[dst_v.at[jv]], sem, add=True)

    def s_wait(buf, sem):
        pltpu.make_async_copy(buf, acc_sh.at[dst_v.at[0]], sem).wait()

    def chunk(ch, _):
        pltpu.sync_copy(src_hbm.at[c, s, ch], src_v)
        pltpu.sync_copy(dst_hbm.at[c, s, ch], dst_v)
        g_start(0, gbuf0, gsem0)
        g_start(1, gbuf1, gsem1)

        def pair(kk, _):
            j = 2 * kk
            g_wait(gbuf0, gsem0)
            s_start(j, gbuf0, ssem0)
            g_wait(gbuf1, gsem1)
            s_start(j + 1, gbuf1, ssem1)
            s_wait(gbuf0, ssem0)
            g_start(j + 2, gbuf0, gsem0)
            s_wait(gbuf1, ssem1)
            g_start(j + 3, gbuf1, gsem1)
            return 0

        lax.fori_loop(0, NB_CH // 2 - 1, pair, 0)
        g_wait(gbuf0, gsem0)
        s_start(NB_CH - 2, gbuf0, ssem0)
        g_wait(gbuf1, gsem1)
        s_start(NB_CH - 1, gbuf1, ssem1)
        s_wait(gbuf0, ssem0)
        s_wait(gbuf1, ssem1)
        return 0

    lax.fori_loop(0, NCHUNK, chunk, 0)
    plsc.subcore_barrier()
    _stage(s, acc_sh, out_hbm.at[c])


@functools.cache
def _agg_call():
    return pl.kernel(
        _agg_body,
        out_type=jax.ShapeDtypeStruct((NC, N_NODES, D_FEAT), jnp.float32),
        mesh=_mesh(),
        scratch_types=(
            [pltpu.VMEM((NB_CH, KBLK), jnp.int32)] * 2
            + [pltpu.VMEM((KBLK, D_FEAT), jnp.float32)] * 2
            + [pltpu.MemorySpace.VMEM_SHARED((N_PAD, D_FEAT), jnp.float32)]
            + [pltpu.SemaphoreType.DMA] * 4
        ),
    )


MBLK = 1000
GRID = N_NODES // MBLK


def _mm1_body(deg_ref, x_ref, w_ref, y_ref, dinv_ref):
    deg = jnp.sum(deg_ref[...], axis=0) + 1.0
    dinv = lax.rsqrt(deg)
    xw = jnp.dot(x_ref[...], w_ref[...], preferred_element_type=jnp.float32)
    y_ref[...] = xw * dinv
    dinv_ref[...] = dinv


def _mid_body(agg_ref, y_ref, dinv_ref, b1_ref, w_ref, out_ref):
    a = agg_ref[0] + agg_ref[1] - y_ref[...]
    dinv = dinv_ref[...]
    h = jnp.maximum(a * dinv + b1_ref[...], 0.0)
    out_ref[...] = jnp.dot(h, w_ref[...], preferred_element_type=jnp.float32) * dinv


def _head_body(agg_ref, y_ref, dinv_ref, b2_ref, w3_ref, b3_ref, out_ref):
    a = agg_ref[0] + agg_ref[1] - y_ref[...]
    h = jnp.maximum(a * dinv_ref[...] + b2_ref[...], 0.0)
    out_ref[...] = (
        jnp.dot(h, w3_ref[...], preferred_element_type=jnp.float32) + b3_ref[...]
    )


def _mm1(degs, x, W1):
    return pl.pallas_call(
        _mm1_body,
        grid=(GRID,),
        in_specs=[
            pl.BlockSpec((NC * NS, MBLK, 1), lambda i: (0, i, 0)),
            pl.BlockSpec((MBLK, D_FEAT), lambda i: (i, 0)),
            pl.BlockSpec((D_FEAT, HIDDEN), lambda i: (0, 0)),
        ],
        out_specs=[
            pl.BlockSpec((MBLK, HIDDEN), lambda i: (i, 0)),
            pl.BlockSpec((MBLK, 1), lambda i: (i, 0)),
        ],
        out_shape=[
            jax.ShapeDtypeStruct((N_NODES, HIDDEN), jnp.float32),
            jax.ShapeDtypeStruct((N_NODES, 1), jnp.float32),
        ],
    )(degs, x, W1)


def _mid(agg1, y1, dinv, b1, W2):
    return pl.pallas_call(
        _mid_body,
        grid=(GRID,),
        in_specs=[
            pl.BlockSpec((NC, MBLK, HIDDEN), lambda i: (0, i, 0)),
            pl.BlockSpec((MBLK, HIDDEN), lambda i: (i, 0)),
            pl.BlockSpec((MBLK, 1), lambda i: (i, 0)),
            pl.BlockSpec((1, HIDDEN), lambda i: (0, 0)),
            pl.BlockSpec((HIDDEN, HIDDEN), lambda i: (0, 0)),
        ],
        out_specs=pl.BlockSpec((MBLK, HIDDEN), lambda i: (i, 0)),
        out_shape=jax.ShapeDtypeStruct((N_NODES, HIDDEN), jnp.float32),
    )(agg1, y1, dinv, b1, W2)


def _head(agg2, y2, dinv, b2, W3, b3):
    return pl.pallas_call(
        _head_body,
        grid=(GRID,),
        in_specs=[
            pl.BlockSpec((NC, MBLK, HIDDEN), lambda i: (0, i, 0)),
            pl.BlockSpec((MBLK, HIDDEN), lambda i: (i, 0)),
            pl.BlockSpec((MBLK, 1), lambda i: (i, 0)),
            pl.BlockSpec((1, HIDDEN), lambda i: (0, 0)),
            pl.BlockSpec((HIDDEN, N_CLASSES), lambda i: (0, 0)),
            pl.BlockSpec((1, N_CLASSES), lambda i: (0, 0)),
        ],
        out_specs=pl.BlockSpec((MBLK, N_CLASSES), lambda i: (i, 0)),
        out_shape=jax.ShapeDtypeStruct((N_NODES, N_CLASSES), jnp.float32),
    )(agg2, y2, dinv, b2, W3, b3)


def kernel(x, edge_index, W1, b1, W2, b2, W3, b3):
    ei = edge_index.astype(jnp.int32)
    trash = N_NODES + (jnp.arange(NC * NS, dtype=jnp.int32) % NS)
    padblk = jnp.broadcast_to(trash[:, None],
                              (NC * NS, EPT_PAD - EDGES_PER_TILE))
    src = jnp.concatenate(
        [ei[0].reshape(NC * NS, EDGES_PER_TILE), padblk], axis=1
    ).reshape(NC, NS, NCHUNK, NB_CH, KBLK)
    dst = jnp.concatenate(
        [ei[1].reshape(NC * NS, EDGES_PER_TILE), padblk], axis=1
    ).reshape(NC, NS, NCHUNK, NB_CH, KBLK)
    dst_deg = ei[1].reshape(NC * NS, EDGES_PER_TILE)

    degs = _deg_call()(dst_deg)
    y1, dinv = _mm1(degs.reshape(NC * NS, N_NODES, 1), x, W1)
    y1p = jnp.pad(y1, ((0, N_PAD - N_NODES), (0, 0)))
    agg1 = _agg_call()(src, dst, y1p)
    y2 = _mid(agg1, y1, dinv, b1.reshape(1, HIDDEN), W2)
    y2p = jnp.pad(y2, ((0, N_PAD - N_NODES), (0, 0)))
    agg2 = _agg_call()(src, dst, y2p)
    logits = _head(agg2, y2, dinv, b2.reshape(1, HIDDEN), W3,
                   b3.reshape(1, N_CLASSES))
    return logits


# speedup vs baseline: 2.5277x; 1.4401x over previous
import functools

import jax
import jax.numpy as jnp
from jax import lax
from jax.experimental import pallas as pl
from jax.experimental.pallas import tpu as pltpu, tpu_sc as plsc

N_NODES = 10000
N_EDGES = 320000
D_FEAT = 128
HIDDEN = 128
N_CLASSES = 40

NC = 2
NS = 16
LANES = 16

EDGES_PER_TILE = N_EDGES // (NC * NS)
KBLK = 40
NBLK = EDGES_PER_TILE // KBLK
NB_CH = 50
NCHUNK = NBLK // NB_CH
CH_EDGES = NB_CH * KBLK


@functools.cache
def _mesh():
    return plsc.VectorSubcoreMesh(
        core_axis_name="c", subcore_axis_name="s", num_cores=NC, num_subcores=NS
    )


def _deg_body(dst_hbm, out_hbm, dst_v, hist_v):
    c = lax.axis_index("c")
    s = lax.axis_index("s")
    w = c * NS + s
    base = pl.multiple_of(w * EDGES_PER_TILE, 8)
    pltpu.sync_copy(dst_hbm.at[pl.ds(base, EDGES_PER_TILE)], dst_v)
    zeros = jnp.zeros((LANES,), jnp.float32)

    def zbody(i, _):
        hist_v[pl.ds(i * LANES, LANES)] = zeros
        return 0

    lax.fori_loop(0, N_NODES // LANES, zbody, 0)
    ones = jnp.ones((LANES,), jnp.float32)

    def body(i, _):
        idx = dst_v[pl.ds(i * LANES, LANES)]
        plsc.addupdate_scatter(hist_v, [idx], ones)
        return 0

    lax.fori_loop(0, EDGES_PER_TILE // LANES, body, 0)
    pltpu.sync_copy(hist_v, out_hbm.at[w])


@functools.cache
def _deg_call():
    return pl.kernel(
        _deg_body,
        out_type=jax.ShapeDtypeStruct((NC * NS, N_NODES), jnp.float32),
        mesh=_mesh(),
        scratch_types=[
            pltpu.VMEM((EDGES_PER_TILE,), jnp.int32),
            pltpu.VMEM((N_NODES,), jnp.float32),
        ],
        compiler_params=pltpu.CompilerParams(needs_layout_passes=False),
    )


RCHUNK = 624
RLAST = N_NODES - (NS - 1) * RCHUNK


def _stage(s, src_view, dst_view):
    r0 = pl.multiple_of(s * RCHUNK, 8)

    @pl.when(s < NS - 1)
    def _():
        pltpu.sync_copy(src_view.at[pl.ds(r0, RCHUNK)],
                        dst_view.at[pl.ds(r0, RCHUNK)])

    @pl.when(s == NS - 1)
    def _():
        pltpu.sync_copy(src_view.at[pl.ds((NS - 1) * RCHUNK, RLAST)],
                        dst_view.at[pl.ds((NS - 1) * RCHUNK, RLAST)])


def _agg_body(src_hbm, dst_hbm, y_hbm, out_hbm, src_v, dstb, gbuf0, gbuf1,
              acc_sh, gsem0, gsem1, ssem0, ssem1, dsem0, dsem1):
    c = lax.axis_index("c")
    s = lax.axis_index("s")
    gbufs = (gbuf0, gbuf1)
    gsems = (gsem0, gsem1)
    ssems = (ssem0, ssem1)
    dsems = (dsem0, dsem1)
    _stage(s, y_hbm, acc_sh)
    plsc.subcore_barrier()

    tile_base = (c * NS + s) * EDGES_PER_TILE

    def chunk(ch, _):
        chbase = pl.multiple_of(tile_base + ch * CH_EDGES, 8)
        pltpu.sync_copy(src_hbm.at[pl.ds(chbase, CH_EDGES)], src_v)

        def d_start(j, b):
            off = pl.multiple_of(chbase + j * KBLK, 8)
            pltpu.async_copy(dst_hbm.at[pl.ds(off, KBLK)],
                             dstb.at[b], dsems[b])

        def d_wait(b):
            pltpu.make_async_copy(dst_hbm.at[pl.ds(0, KBLK)],
                                  dstb.at[b], dsems[b]).wait()

        def g_start(j, b):
            idx = src_v.at[pl.ds(j * KBLK, KBLK)]
            pltpu.async_copy(y_hbm.at[idx], gbufs[b], gsems[b])

        def g_wait(b):
            idx = src_v.at[pl.ds(0, KBLK)]
            pltpu.make_async_copy(y_hbm.at[idx], gbufs[b], gsems[b]).wait()

        def s_start(b):
            pltpu.async_copy(gbufs[b], acc_sh.at[dstb.at[b]], ssems[b],
                             add=True)

        def s_wait(b):
            pltpu.make_async_copy(gbufs[b], acc_sh.at[dstb.at[0]],
                                  ssems[b]).wait()

        d_start(0, 0)
        g_start(0, 0)
        d_start(1, 1)
        g_start(1, 1)

        def pair(kk, _):
            j = 2 * kk
            for b in range(2):
                d_wait(b)
                g_wait(b)
                s_start(b)
                s_wait(b)
                d_start(j + 2 + b, b)
                g_start(j + 2 + b, b)
            return 0

        lax.fori_loop(0, NB_CH // 2 - 1, pair, 0)
        for b in range(2):
            d_wait(b)
            g_wait(b)
            s_start(b)
        for b in range(2):
            s_wait(b)
        return 0

    lax.fori_loop(0, NCHUNK, chunk, 0)
    plsc.subcore_barrier()
    _stage(s, acc_sh, out_hbm.at[c])


@functools.cache
def _agg_call():
    return pl.kernel(
        _agg_body,
        out_type=jax.ShapeDtypeStruct((NC, N_NODES, D_FEAT), jnp.float32),
        mesh=_mesh(),
        scratch_types=(
            [pltpu.VMEM((CH_EDGES,), jnp.int32)]
            + [pltpu.VMEM((2, KBLK), jnp.int32)]
            + [pltpu.VMEM((KBLK, D_FEAT), jnp.float32)] * 2
            + [pltpu.MemorySpace.VMEM_SHARED((N_NODES, D_FEAT), jnp.float32)]
            + [pltpu.SemaphoreType.DMA] * 6
        ),
    )


MBLK = 1000
GRID = N_NODES // MBLK


def _mm1_body(deg_ref, x_ref, w_ref, y_ref, dinv_ref):
    deg = jnp.sum(deg_ref[...], axis=1) + 1.0
    dinv = lax.rsqrt(deg)[:, None]
    xw = jnp.dot(x_ref[...], w_ref[...], preferred_element_type=jnp.float32)
    y_ref[...] = xw * dinv
    dinv_ref[...] = dinv


def _mid_body(agg_ref, y_ref, dinv_ref, b1_ref, w_ref, out_ref):
    a = agg_ref[0] + agg_ref[1] - y_ref[...]
    dinv = dinv_ref[...]
    h = jnp.maximum(a * dinv + b1_ref[...], 0.0)
    out_ref[...] = jnp.dot(h, w_ref[...], preferred_element_type=jnp.float32) * dinv


def _head_body(agg_ref, y_ref, dinv_ref, b2_ref, w3_ref, b3_ref, out_ref):
    a = agg_ref[0] + agg_ref[1] - y_ref[...]
    h = jnp.maximum(a * dinv_ref[...] + b2_ref[...], 0.0)
    out_ref[...] = (
        jnp.dot(h, w3_ref[...], preferred_element_type=jnp.float32) + b3_ref[...]
    )


def _mm1(degs, x, W1):
    return pl.pallas_call(
        _mm1_body,
        grid=(GRID,),
        in_specs=[
            pl.BlockSpec((MBLK, NC * NS), lambda i: (i, 0)),
            pl.BlockSpec((MBLK, D_FEAT), lambda i: (i, 0)),
            pl.BlockSpec((D_FEAT, HIDDEN), lambda i: (0, 0)),
        ],
        out_specs=[
            pl.BlockSpec((MBLK, HIDDEN), lambda i: (i, 0)),
            pl.BlockSpec((MBLK, 1), lambda i: (i, 0)),
        ],
        out_shape=[
            jax.ShapeDtypeStruct((N_NODES, HIDDEN), jnp.float32),
            jax.ShapeDtypeStruct((N_NODES, 1), jnp.float32),
        ],
    )(degs, x, W1)


def _mid(agg1, y1, dinv, b1, W2):
    return pl.pallas_call(
        _mid_body,
        grid=(GRID,),
        in_specs=[
            pl.BlockSpec((NC, MBLK, HIDDEN), lambda i: (0, i, 0)),
            pl.BlockSpec((MBLK, HIDDEN), lambda i: (i, 0)),
            pl.BlockSpec((MBLK, 1), lambda i: (i, 0)),
            pl.BlockSpec((1, HIDDEN), lambda i: (0, 0)),
            pl.BlockSpec((HIDDEN, HIDDEN), lambda i: (0, 0)),
        ],
        out_specs=pl.BlockSpec((MBLK, HIDDEN), lambda i: (i, 0)),
        out_shape=jax.ShapeDtypeStruct((N_NODES, HIDDEN), jnp.float32),
    )(agg1, y1, dinv, b1, W2)


def _head(agg2, y2, dinv, b2, W3, b3):
    return pl.pallas_call(
        _head_body,
        grid=(GRID,),
        in_specs=[
            pl.BlockSpec((NC, MBLK, HIDDEN), lambda i: (0, i, 0)),
            pl.BlockSpec((MBLK, HIDDEN), lambda i: (i, 0)),
            pl.BlockSpec((MBLK, 1), lambda i: (i, 0)),
            pl.BlockSpec((1, HIDDEN), lambda i: (0, 0)),
            pl.BlockSpec((HIDDEN, N_CLASSES), lambda i: (0, 0)),
            pl.BlockSpec((1, N_CLASSES), lambda i: (0, 0)),
        ],
        out_specs=pl.BlockSpec((MBLK, N_CLASSES), lambda i: (i, 0)),
        out_shape=jax.ShapeDtypeStruct((N_NODES, N_CLASSES), jnp.float32),
    )(agg2, y2, dinv, b2, W3, b3)


def kernel(x, edge_index, W1, b1, W2, b2, W3, b3):
    ei = edge_index.astype(jnp.int32)

    src1d = ei[0]
    dst1d = ei[1]
    degs = _deg_call()(dst1d)
    y1, dinv = _mm1(degs.T, x, W1)
    agg1 = _agg_call()(src1d, dst1d, y1)
    y2 = _mid(agg1, y1, dinv, b1.reshape(1, HIDDEN), W2)
    agg2 = _agg_call()(src1d, dst1d, y2)
    logits = _head(agg2, y2, dinv, b2.reshape(1, HIDDEN), W3,
                   b3.reshape(1, N_CLASSES))
    return logits


# speedup vs baseline: 3.1420x; 1.2430x over previous
import functools

import jax
import jax.numpy as jnp
from jax import lax
from jax.experimental import pallas as pl
from jax.experimental.pallas import tpu as pltpu, tpu_sc as plsc

N_NODES = 10000
N_EDGES = 320000
D_FEAT = 128
HIDDEN = 128
N_CLASSES = 40

NC = 2
NS = 16
LANES = 16

EDGES_PER_TILE = N_EDGES // (NC * NS)
KBLK = 40
NBLK = EDGES_PER_TILE // KBLK
NB_CH = 50
NCHUNK = NBLK // NB_CH
CH_EDGES = NB_CH * KBLK


@functools.cache
def _mesh():
    return plsc.VectorSubcoreMesh(
        core_axis_name="c", subcore_axis_name="s", num_cores=NC, num_subcores=NS
    )


def _deg_body(dst_hbm, out_hbm, dst_v, hist_v):
    c = lax.axis_index("c")
    s = lax.axis_index("s")
    w = c * NS + s
    base = pl.multiple_of(w * EDGES_PER_TILE, 8)
    pltpu.sync_copy(dst_hbm.at[pl.ds(base, EDGES_PER_TILE)], dst_v)
    zeros = jnp.zeros((LANES,), jnp.float32)

    def zbody(i, _):
        hist_v[pl.ds(i * LANES, LANES)] = zeros
        return 0

    lax.fori_loop(0, N_NODES // LANES, zbody, 0)
    ones = jnp.ones((LANES,), jnp.float32)

    def body(i, _):
        idx = dst_v[pl.ds(i * LANES, LANES)]
        plsc.addupdate_scatter(hist_v, [idx], ones)
        return 0

    lax.fori_loop(0, EDGES_PER_TILE // LANES, body, 0)
    pltpu.sync_copy(hist_v, out_hbm.at[w])


@functools.cache
def _deg_call():
    return pl.kernel(
        _deg_body,
        out_type=jax.ShapeDtypeStruct((NC * NS, N_NODES), jnp.float32),
        mesh=_mesh(),
        scratch_types=[
            pltpu.VMEM((EDGES_PER_TILE,), jnp.int32),
            pltpu.VMEM((N_NODES,), jnp.float32),
        ],
        compiler_params=pltpu.CompilerParams(needs_layout_passes=False),
    )


RCHUNK = 624
RLAST = N_NODES - (NS - 1) * RCHUNK


def _stage(s, src_view, dst_view):
    r0 = pl.multiple_of(s * RCHUNK, 8)

    @pl.when(s < NS - 1)
    def _():
        pltpu.sync_copy(src_view.at[pl.ds(r0, RCHUNK)],
                        dst_view.at[pl.ds(r0, RCHUNK)])

    @pl.when(s == NS - 1)
    def _():
        pltpu.sync_copy(src_view.at[pl.ds((NS - 1) * RCHUNK, RLAST)],
                        dst_view.at[pl.ds((NS - 1) * RCHUNK, RLAST)])


NBUF = 3
NTRI = 15


def _agg_body(src_hbm, dst_hbm, y_hbm, out_hbm, src_v, dstb,
              gbuf0, gbuf1, gbuf2, acc_sh,
              gsem0, gsem1, gsem2, ssem0, ssem1, ssem2, dsem0, dsem1, dsem2):
    c = lax.axis_index("c")
    s = lax.axis_index("s")
    gbufs = (gbuf0, gbuf1, gbuf2)
    gsems = (gsem0, gsem1, gsem2)
    ssems = (ssem0, ssem1, ssem2)
    dsems = (dsem0, dsem1, dsem2)
    _stage(s, y_hbm, acc_sh)
    plsc.subcore_barrier()

    tile_base = (c * NS + s) * EDGES_PER_TILE

    def chunk(ch, _):
        chbase = pl.multiple_of(tile_base + ch * CH_EDGES, 8)
        pltpu.sync_copy(src_hbm.at[pl.ds(chbase, CH_EDGES)], src_v)

        def d_start(j, b):
            off = pl.multiple_of(chbase + j * KBLK, 8)
            pltpu.async_copy(dst_hbm.at[pl.ds(off, KBLK)],
                             dstb.at[b], dsems[b])

        def d_wait(b):
            pltpu.make_async_copy(dst_hbm.at[pl.ds(0, KBLK)],
                                  dstb.at[b], dsems[b]).wait()

        def g_start(j, b):
            idx = src_v.at[pl.ds(j * KBLK, KBLK)]
            pltpu.async_copy(y_hbm.at[idx], gbufs[b], gsems[b])

        def g_wait(b):
            idx = src_v.at[pl.ds(0, KBLK)]
            pltpu.make_async_copy(y_hbm.at[idx], gbufs[b], gsems[b]).wait()

        def s_start(b):
            pltpu.async_copy(gbufs[b], acc_sh.at[dstb.at[b]], ssems[b],
                             add=True)

        def s_wait(b):
            pltpu.make_async_copy(gbufs[b], acc_sh.at[dstb.at[0]],
                                  ssems[b]).wait()

        for b in range(NBUF):
            d_start(b, b)
            g_start(b, b)

        def triple(t, _):
            j = NBUF * t
            for b in range(NBUF):
                d_wait(b)
                g_wait(b)
                s_start(b)
                s_wait(b)
                d_start(j + NBUF + b, b)
                g_start(j + NBUF + b, b)
            return 0

        lax.fori_loop(0, NTRI, triple, 0)
        for b in range(NBUF):
            d_wait(b)
            g_wait(b)
            s_start(b)
            if b < NB_CH - NBUF * (NTRI + 1):
                s_wait(b)
                d_start(NBUF * (NTRI + 1) + b, b)
                g_start(NBUF * (NTRI + 1) + b, b)
        for b in range(NB_CH - NBUF * (NTRI + 1)):
            d_wait(b)
            g_wait(b)
            s_start(b)
        for b in range(NBUF):
            s_wait(b)
        return 0

    lax.fori_loop(0, NCHUNK, chunk, 0)
    plsc.subcore_barrier()
    _stage(s, acc_sh, out_hbm.at[c])


@functools.cache
def _agg_call():
    return pl.kernel(
        _agg_body,
        out_type=jax.ShapeDtypeStruct((NC, N_NODES, D_FEAT), jnp.float32),
        mesh=_mesh(),
        scratch_types=(
            [pltpu.VMEM((CH_EDGES,), jnp.int32)]
            + [pltpu.VMEM((NBUF, KBLK), jnp.int32)]
            + [pltpu.VMEM((KBLK, D_FEAT), jnp.float32)] * NBUF
            + [pltpu.MemorySpace.VMEM_SHARED((N_NODES, D_FEAT), jnp.float32)]
            + [pltpu.SemaphoreType.DMA] * (3 * NBUF)
        ),
    )


MBLK = 1000
GRID = N_NODES // MBLK


def _mm1_body(deg_ref, x_ref, w_ref, y_ref, dinv_ref):
    deg = jnp.sum(deg_ref[...], axis=1) + 1.0
    dinv = lax.rsqrt(deg)[:, None]
    xw = jnp.dot(x_ref[...], w_ref[...], preferred_element_type=jnp.float32)
    y_ref[...] = xw * dinv
    dinv_ref[...] = dinv


def _mid_body(agg_ref, y_ref, dinv_ref, b1_ref, w_ref, out_ref):
    a = agg_ref[0] + agg_ref[1] - y_ref[...]
    dinv = dinv_ref[...]
    h = jnp.maximum(a * dinv + b1_ref[...], 0.0)
    out_ref[...] = jnp.dot(h, w_ref[...], preferred_element_type=jnp.float32) * dinv


def _head_body(agg_ref, y_ref, dinv_ref, b2_ref, w3_ref, b3_ref, out_ref):
    a = agg_ref[0] + agg_ref[1] - y_ref[...]
    h = jnp.maximum(a * dinv_ref[...] + b2_ref[...], 0.0)
    out_ref[...] = (
        jnp.dot(h, w3_ref[...], preferred_element_type=jnp.float32) + b3_ref[...]
    )


def _mm1(degs, x, W1):
    return pl.pallas_call(
        _mm1_body,
        grid=(GRID,),
        in_specs=[
            pl.BlockSpec((MBLK, NC * NS), lambda i: (i, 0)),
            pl.BlockSpec((MBLK, D_FEAT), lambda i: (i, 0)),
            pl.BlockSpec((D_FEAT, HIDDEN), lambda i: (0, 0)),
        ],
        out_specs=[
            pl.BlockSpec((MBLK, HIDDEN), lambda i: (i, 0)),
            pl.BlockSpec((MBLK, 1), lambda i: (i, 0)),
        ],
        out_shape=[
            jax.ShapeDtypeStruct((N_NODES, HIDDEN), jnp.float32),
            jax.ShapeDtypeStruct((N_NODES, 1), jnp.float32),
        ],
    )(degs, x, W1)


def _mid(agg1, y1, dinv, b1, W2):
    return pl.pallas_call(
        _mid_body,
        grid=(GRID,),
        in_specs=[
            pl.BlockSpec((NC, MBLK, HIDDEN), lambda i: (0, i, 0)),
            pl.BlockSpec((MBLK, HIDDEN), lambda i: (i, 0)),
            pl.BlockSpec((MBLK, 1), lambda i: (i, 0)),
            pl.BlockSpec((1, HIDDEN), lambda i: (0, 0)),
            pl.BlockSpec((HIDDEN, HIDDEN), lambda i: (0, 0)),
        ],
        out_specs=pl.BlockSpec((MBLK, HIDDEN), lambda i: (i, 0)),
        out_shape=jax.ShapeDtypeStruct((N_NODES, HIDDEN), jnp.float32),
    )(agg1, y1, dinv, b1, W2)


def _head(agg2, y2, dinv, b2, W3, b3):
    return pl.pallas_call(
        _head_body,
        grid=(GRID,),
        in_specs=[
            pl.BlockSpec((NC, MBLK, HIDDEN), lambda i: (0, i, 0)),
            pl.BlockSpec((MBLK, HIDDEN), lambda i: (i, 0)),
            pl.BlockSpec((MBLK, 1), lambda i: (i, 0)),
            pl.BlockSpec((1, HIDDEN), lambda i: (0, 0)),
            pl.BlockSpec((HIDDEN, N_CLASSES), lambda i: (0, 0)),
            pl.BlockSpec((1, N_CLASSES), lambda i: (0, 0)),
        ],
        out_specs=pl.BlockSpec((MBLK, N_CLASSES), lambda i: (i, 0)),
        out_shape=jax.ShapeDtypeStruct((N_NODES, N_CLASSES), jnp.float32),
    )(agg2, y2, dinv, b2, W3, b3)


def kernel(x, edge_index, W1, b1, W2, b2, W3, b3):
    ei = edge_index.astype(jnp.int32)

    src1d = ei[0]
    dst1d = ei[1]
    degs = _deg_call()(dst1d)
    y1, dinv = _mm1(degs.T, x, W1)
    agg1 = _agg_call()(src1d, dst1d, y1)
    y2 = _mid(agg1, y1, dinv, b1.reshape(1, HIDDEN), W2)
    agg2 = _agg_call()(src1d, dst1d, y2)
    logits = _head(agg2, y2, dinv, b2.reshape(1, HIDDEN), W3,
                   b3.reshape(1, N_CLASSES))
    return logits
